# Initial kernel scaffold; baseline (speedup 1.0000x reference)
import functools

import jax
import jax.numpy as jnp
from jax import lax
from jax.experimental import pallas as pl
from jax.experimental.pallas import tpu as pltpu
from jax.experimental.pallas import tpu_sc as plsc

N = 10000
D = 128
G = 64
NC = 2
NS = 16
NW = NC * NS
CH = 128
ACC_ROWS = 10016
RPT = ACC_ROWS // NS
RB = 1000
DEGW = 16

_mesh = plsc.VectorSubcoreMesh(core_axis_name="c", subcore_axis_name="s")


@functools.partial(
    pl.kernel,
    mesh=_mesh,
    out_type=jax.ShapeDtypeStruct((NC, ACC_ROWS, DEGW), jnp.float32),
    scratch_types=[
        pltpu.VMEM((CH,), jnp.int32),
        pltpu.VMEM((CH, DEGW), jnp.float32),
        pltpu.VMEM_SHARED((ACC_ROWS, DEGW), jnp.float32),
    ],
)
def _deg_kernel(dst_hbm, zeros_hbm, ones_hbm, out_hbm, dst_v, ones_v, acc_sh):
    c = lax.axis_index("c")
    s = lax.axis_index("s")
    wid = s * NC + c
    epw = dst_hbm.shape[0] // NW
    n_ch = epw // CH
    r0 = s * RPT
    pltpu.sync_copy(zeros_hbm.at[pl.ds(r0, RPT)], acc_sh.at[pl.ds(r0, RPT)])
    pltpu.sync_copy(ones_hbm, ones_v)
    plsc.subcore_barrier()

    def body(i, carry):
        off = wid * epw + i * CH
        pltpu.sync_copy(dst_hbm.at[pl.ds(off, CH)], dst_v)
        pltpu.sync_copy(ones_v, acc_sh.at[dst_v], add=True)
        return carry

    lax.fori_loop(0, n_ch, body, 0)
    plsc.subcore_barrier()
    pltpu.sync_copy(acc_sh.at[pl.ds(r0, RPT)], out_hbm.at[c, pl.ds(r0, RPT)])


@functools.partial(
    pl.kernel,
    mesh=_mesh,
    out_type=jax.ShapeDtypeStruct((NC, ACC_ROWS, D), jnp.float32),
    scratch_types=[
        pltpu.VMEM((CH,), jnp.int32),
        pltpu.VMEM((CH,), jnp.int32),
        pltpu.VMEM((CH, D), jnp.float32),
        pltpu.VMEM_SHARED((ACC_ROWS, D), jnp.float32),
        pltpu.SemaphoreType.DMA,
    ],
)
def _agg_kernel(g_hbm, src_hbm, dst_hbm, zeros_hbm, out_hbm,
                src_v, dst_v, rows_v, acc_sh, sem):
    c = lax.axis_index("c")
    s = lax.axis_index("s")
    wid = s * NC + c
    epw = src_hbm.shape[0] // NW
    n_ch = epw // CH
    r0 = s * RPT
    pltpu.sync_copy(zeros_hbm.at[pl.ds(r0, RPT)], acc_sh.at[pl.ds(r0, RPT)])
    plsc.subcore_barrier()

    def body(i, carry):
        off = wid * epw + i * CH
        pltpu.sync_copy(src_hbm.at[pl.ds(off, CH)], src_v)
        pltpu.sync_copy(dst_hbm.at[pl.ds(off, CH)], dst_v)
        pltpu.async_copy(g_hbm.at[src_v], rows_v, sem).wait()
        pltpu.sync_copy(rows_v, acc_sh.at[dst_v], add=True)
        return carry

    lax.fori_loop(0, n_ch, body, 0)
    plsc.subcore_barrier()
    pltpu.sync_copy(acc_sh.at[pl.ds(r0, RPT)], out_hbm.at[c, pl.ds(r0, RPT)])


def _p_body(deg_ref, x_ref, w_ref, dinv_ref, g_ref):
    deg = deg_ref[0, :, 0:1] + deg_ref[1, :, 0:1] + 1.0
    dinv = lax.rsqrt(deg)
    h = jnp.dot(x_ref[...], w_ref[...], preferred_element_type=jnp.float32)
    dinv_ref[...] = dinv
    g_ref[...] = dinv * h


def _tc_p(deg_parts, x, w1):
    return pl.pallas_call(
        _p_body,
        grid=(N // RB,),
        in_specs=[
            pl.BlockSpec((NC, RB, DEGW), lambda i: (0, i, 0)),
            pl.BlockSpec((RB, D), lambda i: (i, 0)),
            pl.BlockSpec((D, D), lambda i: (0, 0)),
        ],
        out_specs=[
            pl.BlockSpec((RB, 1), lambda i: (i, 0)),
            pl.BlockSpec((RB, D), lambda i: (i, 0)),
        ],
        out_shape=[
            jax.ShapeDtypeStruct((N, 1), jnp.float32),
            jax.ShapeDtypeStruct((N, D), jnp.float32),
        ],
    )(deg_parts, x, w1)


def _a_body(sp_ref, g_ref, dinv_ref, b_ref, w_ref, gn_ref):
    agg = sp_ref[0] + sp_ref[1] + g_ref[...]
    pre = dinv_ref[...] * agg + b_ref[...]
    x2 = jnp.where(pre > 0, pre, jnp.exp(pre) - 1.0)
    gn_ref[...] = dinv_ref[...] * jnp.dot(
        x2, w_ref[...], preferred_element_type=jnp.float32)


def _tc_a(s_parts, g, dinv, b, w):
    return pl.pallas_call(
        _a_body,
        grid=(N // RB,),
        in_specs=[
            pl.BlockSpec((NC, RB, D), lambda i: (0, i, 0)),
            pl.BlockSpec((RB, D), lambda i: (i, 0)),
            pl.BlockSpec((RB, 1), lambda i: (i, 0)),
            pl.BlockSpec((1, D), lambda i: (0, 0)),
            pl.BlockSpec((D, D), lambda i: (0, 0)),
        ],
        out_specs=pl.BlockSpec((RB, D), lambda i: (i, 0)),
        out_shape=jax.ShapeDtypeStruct((N, D), jnp.float32),
    )(s_parts, g, dinv, b, w)


def _f_body(sp_ref, g_ref, dinv_ref, b_ref, batch_ref,
            wc1_ref, bc1_ref, wc2_ref, bc2_ref, out_ref, sums_sc, cnt_sc):
    i = pl.program_id(0)

    @pl.when(i == 0)
    def _init():
        sums_sc[...] = jnp.zeros_like(sums_sc)
        cnt_sc[...] = jnp.zeros_like(cnt_sc)

    agg = sp_ref[0] + sp_ref[1] + g_ref[...]
    pre = dinv_ref[...] * agg + b_ref[...]
    h = jnp.where(pre > 0, pre, jnp.exp(pre) - 1.0)
    oh = (batch_ref[...] == lax.broadcasted_iota(jnp.int32, (1, G), 1))
    oh = oh.astype(jnp.float32)
    sums_sc[...] += lax.dot_general(
        oh, h, (((0,), (0,)), ((), ())), preferred_element_type=jnp.float32)
    cnt_sc[...] += lax.dot_general(
        oh, jnp.ones((RB, 1), jnp.float32), (((0,), (0,)), ((), ())),
        preferred_element_type=jnp.float32)

    @pl.when(i == N // RB - 1)
    def _fin():
        pooled = sums_sc[...] / jnp.maximum(cnt_sc[...], 1.0)
        z = jnp.dot(pooled, wc1_ref[...], preferred_element_type=jnp.float32)
        z = jnp.maximum(z + bc1_ref[...], 0.0)
        o = jnp.dot(z, wc2_ref[...], preferred_element_type=jnp.float32)
        o = o + bc2_ref[...]
        out_ref[...] = 1.0 / (1.0 + jnp.exp(-o))


def _tc_f(s_parts, g, dinv, b, batch2d, wc1, bc1, wc2, bc2):
    hh = wc1.shape[1]
    return pl.pallas_call(
        _f_body,
        grid=(N // RB,),
        in_specs=[
            pl.BlockSpec((NC, RB, D), lambda i: (0, i, 0)),
            pl.BlockSpec((RB, D), lambda i: (i, 0)),
            pl.BlockSpec((RB, 1), lambda i: (i, 0)),
            pl.BlockSpec((1, D), lambda i: (0, 0)),
            pl.BlockSpec((RB, 1), lambda i: (i, 0)),
            pl.BlockSpec((D, hh), lambda i: (0, 0)),
            pl.BlockSpec((1, hh), lambda i: (0, 0)),
            pl.BlockSpec((hh, 1), lambda i: (0, 0)),
            pl.BlockSpec((1, 1), lambda i: (0, 0)),
        ],
        out_specs=pl.BlockSpec((G, 1), lambda i: (0, 0)),
        out_shape=jax.ShapeDtypeStruct((G, 1), jnp.float32),
        scratch_shapes=[
            pltpu.VMEM((G, D), jnp.float32),
            pltpu.VMEM((G, 1), jnp.float32),
        ],
    )(s_parts, g, dinv, b, batch2d, wc1, bc1, wc2, bc2)


def kernel(x, edge_index, batch, W1, b1, W2, b2, W3, b3, Wc1, bc1, Wc2, bc2):
    E = edge_index.shape[1]
    e_pad = ((E + NW * CH - 1) // (NW * CH)) * (NW * CH)
    pad = e_pad - E
    src = jnp.concatenate([edge_index[0], jnp.zeros((pad,), jnp.int32)])
    dst = jnp.concatenate([edge_index[1], jnp.full((pad,), N, jnp.int32)])

    zeros_big = jnp.zeros((ACC_ROWS, D), jnp.float32)
    zeros_deg = jnp.zeros((ACC_ROWS, DEGW), jnp.float32)
    ones_deg = jnp.ones((CH, DEGW), jnp.float32)

    deg_parts = _deg_kernel(dst, zeros_deg, ones_deg)
    dinv, g1 = _tc_p(deg_parts, x, W1)

    s1 = _agg_kernel(g1, src, dst, zeros_big)
    g2 = _tc_a(s1, g1, dinv, b1.reshape(1, D), W2)
    s2 = _agg_kernel(g2, src, dst, zeros_big)
    g3 = _tc_a(s2, g2, dinv, b2.reshape(1, D), W3)
    s3 = _agg_kernel(g3, src, dst, zeros_big)

    return _tc_f(s3, g3, dinv, b3.reshape(1, D), batch.reshape(N, 1),
                 Wc1, bc1.reshape(1, -1), Wc2, bc2.reshape(1, 1))


# speedup vs baseline: 8.8619x; 8.8619x over previous
import functools

import jax
import jax.numpy as jnp
from jax import lax
from jax.experimental import pallas as pl
from jax.experimental.pallas import tpu as pltpu
from jax.experimental.pallas import tpu_sc as plsc

N = 10000
D = 128
G = 64
NC = 2
NS = 16
NW = NC * NS
CH = 128
ACC_ROWS = 10112
RPT = ACC_ROWS // NS
RB = 1000
DEGW = 128

@functools.cache
def _get_deg_kernel():
    mesh = plsc.VectorSubcoreMesh(core_axis_name="c", subcore_axis_name="s")

    @functools.partial(
        pl.kernel,
        mesh=mesh,
        out_type=jax.ShapeDtypeStruct((NC, ACC_ROWS, DEGW), jnp.float32),
        scratch_types=[
            pltpu.VMEM((CH,), jnp.int32),
            pltpu.VMEM((CH, DEGW), jnp.float32),
            pltpu.VMEM_SHARED((ACC_ROWS, DEGW), jnp.float32),
        ],
    )
    def deg_kernel(dst_hbm, zeros_hbm, ones_hbm, out_hbm, dst_v, ones_v, acc_sh):
        c = lax.axis_index("c")
        s = lax.axis_index("s")
        wid = s * NC + c
        epw = dst_hbm.shape[0] // NW
        n_ch = epw // CH
        r0 = s * RPT
        pltpu.sync_copy(zeros_hbm.at[pl.ds(r0, RPT)], acc_sh.at[pl.ds(r0, RPT)])
        pltpu.sync_copy(ones_hbm, ones_v)
        plsc.subcore_barrier()

        def body(i, carry):
            off = wid * epw + i * CH
            pltpu.sync_copy(dst_hbm.at[pl.ds(off, CH)], dst_v)
            pltpu.sync_copy(ones_v, acc_sh.at[dst_v], add=True)
            return carry

        lax.fori_loop(0, n_ch, body, 0)
        plsc.subcore_barrier()
        pltpu.sync_copy(acc_sh.at[pl.ds(r0, RPT)], out_hbm.at[c, pl.ds(r0, RPT)])

    return deg_kernel


@functools.cache
def _get_agg_kernel():
    mesh = plsc.VectorSubcoreMesh(core_axis_name="c", subcore_axis_name="s")

    @functools.partial(
        pl.kernel,
        mesh=mesh,
        out_type=jax.ShapeDtypeStruct((NC, ACC_ROWS, D), jnp.float32),
        scratch_types=[
            pltpu.VMEM((CH,), jnp.int32),
            pltpu.VMEM((CH,), jnp.int32),
            pltpu.VMEM((CH, D), jnp.float32),
            pltpu.VMEM_SHARED((ACC_ROWS, D), jnp.float32),
            pltpu.SemaphoreType.DMA,
        ],
    )
    def agg_kernel(g_hbm, src_hbm, dst_hbm, zeros_hbm, out_hbm,
                   src_v, dst_v, rows_v, acc_sh, sem):
        c = lax.axis_index("c")
        s = lax.axis_index("s")
        wid = s * NC + c
        epw = src_hbm.shape[0] // NW
        n_ch = epw // CH
        r0 = s * RPT
        pltpu.sync_copy(zeros_hbm.at[pl.ds(r0, RPT)], acc_sh.at[pl.ds(r0, RPT)])
        plsc.subcore_barrier()

        def body(i, carry):
            off = wid * epw + i * CH
            pltpu.sync_copy(src_hbm.at[pl.ds(off, CH)], src_v)
            pltpu.sync_copy(dst_hbm.at[pl.ds(off, CH)], dst_v)
            pltpu.async_copy(g_hbm.at[src_v], rows_v, sem).wait()
            pltpu.sync_copy(rows_v, acc_sh.at[dst_v], add=True)
            return carry

        lax.fori_loop(0, n_ch, body, 0)
        plsc.subcore_barrier()
        pltpu.sync_copy(acc_sh.at[pl.ds(r0, RPT)], out_hbm.at[c, pl.ds(r0, RPT)])

    return agg_kernel


def _p_body(deg_ref, x_ref, w_ref, dinv_ref, g_ref):
    deg = deg_ref[0, :, 0:1] + deg_ref[1, :, 0:1] + 1.0
    dinv = lax.rsqrt(deg)
    h = jnp.dot(x_ref[...], w_ref[...], preferred_element_type=jnp.float32)
    dinv_ref[...] = dinv
    g_ref[...] = dinv * h


def _tc_p(deg_parts, x, w1):
    return pl.pallas_call(
        _p_body,
        grid=(N // RB,),
        in_specs=[
            pl.BlockSpec((NC, RB, DEGW), lambda i: (0, i, 0)),
            pl.BlockSpec((RB, D), lambda i: (i, 0)),
            pl.BlockSpec((D, D), lambda i: (0, 0)),
        ],
        out_specs=[
            pl.BlockSpec((RB, 1), lambda i: (i, 0)),
            pl.BlockSpec((RB, D), lambda i: (i, 0)),
        ],
        out_shape=[
            jax.ShapeDtypeStruct((N, 1), jnp.float32),
            jax.ShapeDtypeStruct((N, D), jnp.float32),
        ],
    )(deg_parts, x, w1)


def _a_body(sp_ref, g_ref, dinv_ref, b_ref, w_ref, gn_ref):
    agg = sp_ref[0] + sp_ref[1] + g_ref[...]
    pre = dinv_ref[...] * agg + b_ref[...]
    x2 = jnp.where(pre > 0, pre, jnp.exp(pre) - 1.0)
    gn_ref[...] = dinv_ref[...] * jnp.dot(
        x2, w_ref[...], preferred_element_type=jnp.float32)


def _tc_a(s_parts, g, dinv, b, w):
    return pl.pallas_call(
        _a_body,
        grid=(N // RB,),
        in_specs=[
            pl.BlockSpec((NC, RB, D), lambda i: (0, i, 0)),
            pl.BlockSpec((RB, D), lambda i: (i, 0)),
            pl.BlockSpec((RB, 1), lambda i: (i, 0)),
            pl.BlockSpec((1, D), lambda i: (0, 0)),
            pl.BlockSpec((D, D), lambda i: (0, 0)),
        ],
        out_specs=pl.BlockSpec((RB, D), lambda i: (i, 0)),
        out_shape=jax.ShapeDtypeStruct((N, D), jnp.float32),
    )(s_parts, g, dinv, b, w)


def _f_body(sp_ref, g_ref, dinv_ref, b_ref, batch_ref,
            wc1_ref, bc1_ref, wc2_ref, bc2_ref, out_ref, sums_sc, cnt_sc):
    i = pl.program_id(0)

    @pl.when(i == 0)
    def _init():
        sums_sc[...] = jnp.zeros_like(sums_sc)
        cnt_sc[...] = jnp.zeros_like(cnt_sc)

    agg = sp_ref[0] + sp_ref[1] + g_ref[...]
    pre = dinv_ref[...] * agg + b_ref[...]
    h = jnp.where(pre > 0, pre, jnp.exp(pre) - 1.0)
    oh = (batch_ref[...] == lax.broadcasted_iota(jnp.int32, (1, G), 1))
    oh = oh.astype(jnp.float32)
    sums_sc[...] += lax.dot_general(
        oh, h, (((0,), (0,)), ((), ())), preferred_element_type=jnp.float32)
    cnt_sc[...] += lax.dot_general(
        oh, jnp.ones((RB, 1), jnp.float32), (((0,), (0,)), ((), ())),
        preferred_element_type=jnp.float32)

    @pl.when(i == N // RB - 1)
    def _fin():
        pooled = sums_sc[...] / jnp.maximum(cnt_sc[...], 1.0)
        z = jnp.dot(pooled, wc1_ref[...], preferred_element_type=jnp.float32)
        z = jnp.maximum(z + bc1_ref[...], 0.0)
        o = jnp.dot(z, wc2_ref[...], preferred_element_type=jnp.float32)
        o = o + bc2_ref[...]
        out_ref[...] = 1.0 / (1.0 + jnp.exp(-o))


def _tc_f(s_parts, g, dinv, b, batch2d, wc1, bc1, wc2, bc2):
    hh = wc1.shape[1]
    return pl.pallas_call(
        _f_body,
        grid=(N // RB,),
        in_specs=[
            pl.BlockSpec((NC, RB, D), lambda i: (0, i, 0)),
            pl.BlockSpec((RB, D), lambda i: (i, 0)),
            pl.BlockSpec((RB, 1), lambda i: (i, 0)),
            pl.BlockSpec((1, D), lambda i: (0, 0)),
            pl.BlockSpec((RB, 1), lambda i: (i, 0)),
            pl.BlockSpec((D, hh), lambda i: (0, 0)),
            pl.BlockSpec((1, hh), lambda i: (0, 0)),
            pl.BlockSpec((hh, 1), lambda i: (0, 0)),
            pl.BlockSpec((1, 1), lambda i: (0, 0)),
        ],
        out_specs=pl.BlockSpec((G, 1), lambda i: (0, 0)),
        out_shape=jax.ShapeDtypeStruct((G, 1), jnp.float32),
        scratch_shapes=[
            pltpu.VMEM((G, D), jnp.float32),
            pltpu.VMEM((G, 1), jnp.float32),
        ],
    )(s_parts, g, dinv, b, batch2d, wc1, bc1, wc2, bc2)


def kernel(x, edge_index, batch, W1, b1, W2, b2, W3, b3, Wc1, bc1, Wc2, bc2):
    E = edge_index.shape[1]
    e_pad = ((E + NW * CH - 1) // (NW * CH)) * (NW * CH)
    pad = e_pad - E
    src = jnp.concatenate([edge_index[0], jnp.zeros((pad,), jnp.int32)])
    dst = jnp.concatenate([edge_index[1], jnp.full((pad,), N, jnp.int32)])

    zeros_big = jnp.zeros((ACC_ROWS, D), jnp.float32)
    zeros_deg = jnp.zeros((ACC_ROWS, DEGW), jnp.float32)
    ones_deg = jnp.ones((CH, DEGW), jnp.float32)

    deg_kernel = _get_deg_kernel()
    agg_kernel = _get_agg_kernel()

    deg_parts = deg_kernel(dst, zeros_deg, ones_deg)
    dinv, g1 = _tc_p(deg_parts, x, W1)

    s1 = agg_kernel(g1, src, dst, zeros_big)
    g2 = _tc_a(s1, g1, dinv, b1.reshape(1, D), W2)
    s2 = agg_kernel(g2, src, dst, zeros_big)
    g3 = _tc_a(s2, g2, dinv, b2.reshape(1, D), W3)
    s3 = agg_kernel(g3, src, dst, zeros_big)

    return _tc_f(s3, g3, dinv, b3.reshape(1, D), batch.reshape(N, 1),
                 Wc1, bc1.reshape(1, -1), Wc2, bc2.reshape(1, 1))


# speedup vs baseline: 9.5819x; 1.0812x over previous
import functools

import jax
import jax.numpy as jnp
from jax import lax
from jax.experimental import pallas as pl
from jax.experimental.pallas import tpu as pltpu
from jax.experimental.pallas import tpu_sc as plsc

N = 10000
D = 128
G = 64
NC = 2
NS = 16
NW = NC * NS
CH = 128
ACC_ROWS = 10112
RPT = ACC_ROWS // NS
RB = 1000
DEGW = 128

WIN = 16


@functools.cache
def _get_deg_kernel(n_ch):
    mesh = plsc.VectorSubcoreMesh(core_axis_name="c", subcore_axis_name="s")

    @functools.partial(
        pl.kernel,
        mesh=mesh,
        out_type=jax.ShapeDtypeStruct((NC, ACC_ROWS, DEGW), jnp.float32),
        scratch_types=[
            pltpu.VMEM((n_ch, CH), jnp.int32),
            pltpu.VMEM((CH, DEGW), jnp.float32),
            pltpu.VMEM_SHARED((ACC_ROWS, DEGW), jnp.float32),
        ],
    )
    def deg_kernel(dst_hbm, zeros_hbm, ones_hbm, out_hbm, dst_v, ones_v, acc_sh):
        c = lax.axis_index("c")
        s = lax.axis_index("s")
        wid = s * NC + c
        r0 = s * RPT
        pltpu.sync_copy(zeros_hbm.at[pl.ds(r0, RPT)], acc_sh.at[pl.ds(r0, RPT)])
        pltpu.sync_copy(dst_hbm.at[wid], dst_v)
        pltpu.sync_copy(ones_hbm, ones_v)
        plsc.subcore_barrier()

        def body(j, carry):
            pltpu.sync_copy(ones_v, acc_sh.at[dst_v.at[j]], add=True)
            return carry

        lax.fori_loop(0, n_ch, body, 0)
        plsc.subcore_barrier()
        pltpu.sync_copy(acc_sh.at[pl.ds(r0, RPT)], out_hbm.at[c, pl.ds(r0, RPT)])

    return deg_kernel


@functools.cache
def _get_agg_kernel(n_ch):
    n_win = n_ch // WIN
    mesh = plsc.VectorSubcoreMesh(core_axis_name="c", subcore_axis_name="s")

    @functools.partial(
        pl.kernel,
        mesh=mesh,
        out_type=jax.ShapeDtypeStruct((NC, ACC_ROWS, D), jnp.float32),
        scratch_types=[
            pltpu.VMEM((2, WIN, CH), jnp.int32),
            pltpu.VMEM((2, WIN, CH), jnp.int32),
            pltpu.VMEM((CH, D), jnp.float32),
            pltpu.VMEM((CH, D), jnp.float32),
            pltpu.VMEM_SHARED((ACC_ROWS, D), jnp.float32),
            pltpu.SemaphoreType.DMA,
            pltpu.SemaphoreType.DMA,
            pltpu.SemaphoreType.DMA,
        ],
    )
    def agg_kernel(g_hbm, src_hbm, dst_hbm, zeros_hbm, out_hbm,
                   srcw, dstw, rb0, rb1, acc_sh, sg0, sg1, semi):
        rows = (rb0, rb1)
        semg = (sg0, sg1)
        c = lax.axis_index("c")
        s = lax.axis_index("s")
        wid = s * NC + c
        r0 = s * RPT
        pltpu.sync_copy(zeros_hbm.at[pl.ds(r0, RPT)], acc_sh.at[pl.ds(r0, RPT)])
        pltpu.sync_copy(src_hbm.at[wid, pl.ds(0, WIN)], srcw.at[0])
        pltpu.sync_copy(dst_hbm.at[wid, pl.ds(0, WIN)], dstw.at[0])
        pltpu.async_copy(src_hbm.at[wid, pl.ds(WIN, WIN)], srcw.at[1], semi)
        pltpu.async_copy(dst_hbm.at[wid, pl.ds(WIN, WIN)], dstw.at[1], semi)
        plsc.subcore_barrier()
        pltpu.async_copy(g_hbm.at[srcw.at[0, 0]], rows[0], semg[0])
        pltpu.async_copy(g_hbm.at[srcw.at[0, 1]], rows[1], semg[1])

        def outer(o, carry):
            slot = lax.rem(o, 2)
            nslot = 1 - slot
            @pl.when(jnp.logical_and(o > 0, o + 1 < n_win))
            def _prefetch():
                woff = (o + 1) * WIN
                pltpu.async_copy(
                    src_hbm.at[wid, pl.ds(woff, WIN)], srcw.at[nslot], semi)
                pltpu.async_copy(
                    dst_hbm.at[wid, pl.ds(woff, WIN)], dstw.at[nslot], semi)

            for k in range(WIN):
                j = o * WIN + k
                b = k % 2
                pltpu.make_async_copy(
                    g_hbm.at[srcw.at[0, 0]], rows[b], semg[b]).wait()
                pltpu.sync_copy(rows[b], acc_sh.at[dstw.at[slot, k]], add=True)
                if k == WIN - 2:
                    @pl.when(o + 1 < n_win)
                    def _wait_idx():
                        pltpu.make_async_copy(
                            src_hbm.at[wid, pl.ds(0, WIN)], srcw.at[0],
                            semi).wait()
                        pltpu.make_async_copy(
                            dst_hbm.at[wid, pl.ds(0, WIN)], dstw.at[0],
                            semi).wait()
                if k + 2 < WIN:
                    pltpu.async_copy(
                        g_hbm.at[srcw.at[slot, k + 2]], rows[b], semg[b])
                else:
                    @pl.when(o + 1 < n_win)
                    def _gather_next_win():
                        pltpu.async_copy(
                            g_hbm.at[srcw.at[nslot, k + 2 - WIN]],
                            rows[b], semg[b])
            return carry

        lax.fori_loop(0, n_win, outer, 0)
        plsc.subcore_barrier()
        pltpu.sync_copy(acc_sh.at[pl.ds(r0, RPT)], out_hbm.at[c, pl.ds(r0, RPT)])

    return agg_kernel


def _p_body(deg_ref, x_ref, w_ref, dinv_ref, g_ref):
    deg = deg_ref[0, :, 0:1] + deg_ref[1, :, 0:1] + 1.0
    dinv = lax.rsqrt(deg)
    h = jnp.dot(x_ref[...], w_ref[...], preferred_element_type=jnp.float32)
    dinv_ref[...] = dinv
    g_ref[...] = dinv * h


def _tc_p(deg_parts, x, w1):
    return pl.pallas_call(
        _p_body,
        grid=(N // RB,),
        in_specs=[
            pl.BlockSpec((NC, RB, DEGW), lambda i: (0, i, 0)),
            pl.BlockSpec((RB, D), lambda i: (i, 0)),
            pl.BlockSpec((D, D), lambda i: (0, 0)),
        ],
        out_specs=[
            pl.BlockSpec((RB, 1), lambda i: (i, 0)),
            pl.BlockSpec((RB, D), lambda i: (i, 0)),
        ],
        out_shape=[
            jax.ShapeDtypeStruct((N, 1), jnp.float32),
            jax.ShapeDtypeStruct((N, D), jnp.float32),
        ],
    )(deg_parts, x, w1)


def _a_body(sp_ref, g_ref, dinv_ref, b_ref, w_ref, gn_ref):
    agg = sp_ref[0] + sp_ref[1] + g_ref[...]
    pre = dinv_ref[...] * agg + b_ref[...]
    x2 = jnp.where(pre > 0, pre, jnp.exp(pre) - 1.0)
    gn_ref[...] = dinv_ref[...] * jnp.dot(
        x2, w_ref[...], preferred_element_type=jnp.float32)


def _tc_a(s_parts, g, dinv, b, w):
    return pl.pallas_call(
        _a_body,
        grid=(N // RB,),
        in_specs=[
            pl.BlockSpec((NC, RB, D), lambda i: (0, i, 0)),
            pl.BlockSpec((RB, D), lambda i: (i, 0)),
            pl.BlockSpec((RB, 1), lambda i: (i, 0)),
            pl.BlockSpec((1, D), lambda i: (0, 0)),
            pl.BlockSpec((D, D), lambda i: (0, 0)),
        ],
        out_specs=pl.BlockSpec((RB, D), lambda i: (i, 0)),
        out_shape=jax.ShapeDtypeStruct((N, D), jnp.float32),
    )(s_parts, g, dinv, b, w)


def _f_body(sp_ref, g_ref, dinv_ref, b_ref, batch_ref,
            wc1_ref, bc1_ref, wc2_ref, bc2_ref, out_ref, sums_sc, cnt_sc):
    i = pl.program_id(0)

    @pl.when(i == 0)
    def _init():
        sums_sc[...] = jnp.zeros_like(sums_sc)
        cnt_sc[...] = jnp.zeros_like(cnt_sc)

    agg = sp_ref[0] + sp_ref[1] + g_ref[...]
    pre = dinv_ref[...] * agg + b_ref[...]
    h = jnp.where(pre > 0, pre, jnp.exp(pre) - 1.0)
    oh = (batch_ref[...] == lax.broadcasted_iota(jnp.int32, (1, G), 1))
    oh = oh.astype(jnp.float32)
    sums_sc[...] += lax.dot_general(
        oh, h, (((0,), (0,)), ((), ())), preferred_element_type=jnp.float32)
    cnt_sc[...] += lax.dot_general(
        oh, jnp.ones((RB, 1), jnp.float32), (((0,), (0,)), ((), ())),
        preferred_element_type=jnp.float32)

    @pl.when(i == N // RB - 1)
    def _fin():
        pooled = sums_sc[...] / jnp.maximum(cnt_sc[...], 1.0)
        z = jnp.dot(pooled, wc1_ref[...], preferred_element_type=jnp.float32)
        z = jnp.maximum(z + bc1_ref[...], 0.0)
        o = jnp.dot(z, wc2_ref[...], preferred_element_type=jnp.float32)
        o = o + bc2_ref[...]
        out_ref[...] = 1.0 / (1.0 + jnp.exp(-o))


def _tc_f(s_parts, g, dinv, b, batch2d, wc1, bc1, wc2, bc2):
    hh = wc1.shape[1]
    return pl.pallas_call(
        _f_body,
        grid=(N // RB,),
        in_specs=[
            pl.BlockSpec((NC, RB, D), lambda i: (0, i, 0)),
            pl.BlockSpec((RB, D), lambda i: (i, 0)),
            pl.BlockSpec((RB, 1), lambda i: (i, 0)),
            pl.BlockSpec((1, D), lambda i: (0, 0)),
            pl.BlockSpec((RB, 1), lambda i: (i, 0)),
            pl.BlockSpec((D, hh), lambda i: (0, 0)),
            pl.BlockSpec((1, hh), lambda i: (0, 0)),
            pl.BlockSpec((hh, 1), lambda i: (0, 0)),
            pl.BlockSpec((1, 1), lambda i: (0, 0)),
        ],
        out_specs=pl.BlockSpec((G, 1), lambda i: (0, 0)),
        out_shape=jax.ShapeDtypeStruct((G, 1), jnp.float32),
        scratch_shapes=[
            pltpu.VMEM((G, D), jnp.float32),
            pltpu.VMEM((G, 1), jnp.float32),
        ],
    )(s_parts, g, dinv, b, batch2d, wc1, bc1, wc2, bc2)


def kernel(x, edge_index, batch, W1, b1, W2, b2, W3, b3, Wc1, bc1, Wc2, bc2):
    E = edge_index.shape[1]
    quantum = NW * CH * WIN
    e_pad = ((E + quantum - 1) // quantum) * quantum
    n_ch = e_pad // (NW * CH)
    pad = e_pad - E
    src = jnp.concatenate([edge_index[0], jnp.zeros((pad,), jnp.int32)])
    dst = jnp.concatenate([edge_index[1], jnp.full((pad,), N, jnp.int32)])
    src = src.reshape(NW, n_ch, CH)
    dst = dst.reshape(NW, n_ch, CH)

    zeros_big = jnp.zeros((ACC_ROWS, D), jnp.float32)
    zeros_deg = jnp.zeros((ACC_ROWS, DEGW), jnp.float32)
    ones_deg = jnp.ones((CH, DEGW), jnp.float32)

    deg_kernel = _get_deg_kernel(n_ch)
    agg_kernel = _get_agg_kernel(n_ch)

    deg_parts = deg_kernel(dst, zeros_deg, ones_deg)
    dinv, g1 = _tc_p(deg_parts, x, W1)

    s1 = agg_kernel(g1, src, dst, zeros_big)
    g2 = _tc_a(s1, g1, dinv, b1.reshape(1, D), W2)
    s2 = agg_kernel(g2, src, dst, zeros_big)
    g3 = _tc_a(s2, g2, dinv, b2.reshape(1, D), W3)
    s3 = agg_kernel(g3, src, dst, zeros_big)

    return _tc_f(s3, g3, dinv, b3.reshape(1, D), batch.reshape(N, 1),
                 Wc1, bc1.reshape(1, -1), Wc2, bc2.reshape(1, 1))
